# baseline (speedup 1.0000x reference)
import functools
import jax
import jax.numpy as jnp
from jax import lax
from jax.experimental import pallas as pl
from jax.experimental.pallas import tpu as pltpu
from jax.experimental.pallas import tpu_sc as plsc

NC = 2
NS = 16
NW = NC * NS
LANES = 16


def _split_window(chunk):
    for w, ws in ((10000, 2880), (20000, 5120), (8192, 2048), (16, 0)):
        if chunk % w == 0 and (w - ws) % LANES == 0 and ((w - ws) // LANES) % 2 == 1:
            return w, ws
    return chunk, 0


def _sc_partials(idx, vals, num_segments):
    n = vals.shape[0]
    chunk = n // NW
    assert chunk * NW == n
    w, ws = _split_window(chunk)
    nwin = chunk // w
    wr = w - ws
    sub = wr // LANES
    assert num_segments % LANES == 0

    mesh = plsc.VectorSubcoreMesh(
        core_axis_name="c", subcore_axis_name="s", num_cores=NC, num_subcores=NS
    )

    @functools.partial(
        pl.kernel,
        out_type=jax.ShapeDtypeStruct((NW + NC, num_segments), jnp.float32),
        mesh=mesh,
        scratch_types=[
            pltpu.VMEM((ws,), jnp.int32),
            pltpu.VMEM((ws,), jnp.int32),
            pltpu.VMEM((ws,), jnp.float32),
            pltpu.VMEM((ws,), jnp.float32),
            pltpu.VMEM((wr,), jnp.int32),
            pltpu.VMEM((wr,), jnp.int32),
            pltpu.VMEM((wr,), jnp.float32),
            pltpu.VMEM((wr,), jnp.float32),
            pltpu.VMEM((num_segments,), jnp.float32),
            pltpu.VMEM_SHARED((num_segments,), jnp.float32),
            pltpu.SemaphoreType.DMA,
            pltpu.SemaphoreType.DMA,
            pltpu.SemaphoreType.DMA,
            pltpu.SemaphoreType.DMA,
            pltpu.SemaphoreType.DMA,
            pltpu.SemaphoreType.DMA,
        ],
        compiler_params=pltpu.CompilerParams(needs_layout_passes=False),
    )
    def sc_kernel(idx_hbm, val_hbm, part_hbm,
                  sidx0, sidx1, sval0, sval1,
                  ridx0, ridx1, rval0, rval1,
                  acc, spacc,
                  sem_in0, sem_in1, sem_sidx0, sem_sidx1, sem_st0, sem_st1):
        cid = lax.axis_index("c")
        sid = lax.axis_index("s")
        wid = sid * NC + cid
        base = pl.multiple_of(wid * chunk, 8)

        def zero_body(i, _):
            acc[pl.ds(i * LANES, LANES)] = jnp.zeros((LANES,), jnp.float32)
            return 0

        lax.fori_loop(0, num_segments // LANES, zero_body, 0)

        sidx_bufs = (sidx0, sidx1)
        sval_bufs = (sval0, sval1)
        ridx_bufs = (ridx0, ridx1)
        rval_bufs = (rval0, rval1)
        in_sems = (sem_in0, sem_in1)
        sidx_sems = (sem_sidx0, sem_sidx1)
        st_sems = (sem_st0, sem_st1)

        def start_in(widx):
            p = widx % 2
            wb = pl.multiple_of(base + widx * w, 8)
            wbr = pl.multiple_of(wb + ws, 8)
            d = [
                pltpu.async_copy(idx_hbm.at[pl.ds(wb, ws)], sidx_bufs[p], sidx_sems[p]),
                pltpu.async_copy(val_hbm.at[pl.ds(wb, ws)], sval_bufs[p], in_sems[p]),
                pltpu.async_copy(idx_hbm.at[pl.ds(wbr, wr)], ridx_bufs[p], in_sems[p]),
                pltpu.async_copy(val_hbm.at[pl.ds(wbr, wr)], rval_bufs[p], in_sems[p]),
            ]
            return d

        @pl.when(sid == 0)
        def _():
            pltpu.sync_copy(acc, spacc)

        pending = start_in(0)
        plsc.subcore_barrier()

        pos0 = lax.iota(jnp.int32, LANES) * sub
        stream_descs = [None] * nwin

        for widx in range(nwin):
            p = widx % 2
            for d in pending:
                d.wait()
            stream_descs[widx] = pltpu.async_copy(
                sval_bufs[p], spacc.at[sidx_bufs[p]], st_sems[p], add=True
            )
            if widx + 1 < nwin:
                if widx - 1 >= 0:
                    stream_descs[widx - 1].wait()
                pending = start_in(widx + 1)
            idxb = ridx_bufs[p]
            valb = rval_bufs[p]

            @plsc.parallel_loop(0, sub, unroll=8)
            def _(t):
                pos = pos0 + t
                iv = plsc.load_gather(idxb, [pos])
                vv = plsc.load_gather(valb, [pos])
                plsc.addupdate_scatter(acc, [iv], vv)

        if nwin >= 2:
            stream_descs[nwin - 2].wait()
        stream_descs[nwin - 1].wait()
        pltpu.sync_copy(acc, part_hbm.at[wid])
        plsc.subcore_barrier()

        @pl.when(sid == 0)
        def _():
            pltpu.sync_copy(spacc, part_hbm.at[NW + cid])

    return sc_kernel(idx, vals)


def _tc_combine(tensor2d, partials):
    def body(t_ref, p_ref, o_ref):
        o_ref[...] = t_ref[...] - jnp.sum(p_ref[...], axis=0, keepdims=True)

    return pl.pallas_call(
        body,
        out_shape=jax.ShapeDtypeStruct(tensor2d.shape, tensor2d.dtype),
    )(tensor2d, partials)


def kernel(tensor, batch_idx, atomic_numbers, element_references):
    num_segments = tensor.shape[0]
    idx = batch_idx.astype(jnp.int32)
    vals = atomic_numbers.astype(jnp.float32)
    partials = _sc_partials(idx, vals, num_segments)
    out2d = _tc_combine(tensor.reshape(1, -1), partials)
    return out2d.reshape(tensor.shape)


# speedup vs baseline: 1.0703x; 1.0703x over previous
import functools
import jax
import jax.numpy as jnp
from jax import lax
from jax.experimental import pallas as pl
from jax.experimental.pallas import tpu as pltpu
from jax.experimental.pallas import tpu_sc as plsc

NC = 2
NS = 16
NW = NC * NS
LANES = 16


def _split_window(chunk):
    for w, ws in ((10000, 2240), (20000, 5120), (8192, 2048), (16, 0)):
        if chunk % w == 0 and (w - ws) % LANES == 0 and ((w - ws) // LANES) % 2 == 1:
            return w, ws
    return chunk, 0


def _sc_partials(idx, vals, num_segments):
    n = vals.shape[0]
    chunk = n // NW
    assert chunk * NW == n
    w, ws = _split_window(chunk)
    nwin = chunk // w
    wr = w - ws
    sub = wr // LANES
    assert num_segments % LANES == 0

    mesh = plsc.VectorSubcoreMesh(
        core_axis_name="c", subcore_axis_name="s", num_cores=NC, num_subcores=NS
    )

    @functools.partial(
        pl.kernel,
        out_type=jax.ShapeDtypeStruct((NW + NC, num_segments), jnp.float32),
        mesh=mesh,
        scratch_types=[
            pltpu.VMEM((ws,), jnp.int32),
            pltpu.VMEM((ws,), jnp.int32),
            pltpu.VMEM((ws,), jnp.float32),
            pltpu.VMEM((ws,), jnp.float32),
            pltpu.VMEM((wr,), jnp.int32),
            pltpu.VMEM((wr,), jnp.int32),
            pltpu.VMEM((wr,), jnp.float32),
            pltpu.VMEM((wr,), jnp.float32),
            pltpu.VMEM((num_segments,), jnp.float32),
            pltpu.VMEM_SHARED((num_segments,), jnp.float32),
            pltpu.SemaphoreType.DMA,
            pltpu.SemaphoreType.DMA,
            pltpu.SemaphoreType.DMA,
            pltpu.SemaphoreType.DMA,
            pltpu.SemaphoreType.DMA,
            pltpu.SemaphoreType.DMA,
        ],
        compiler_params=pltpu.CompilerParams(needs_layout_passes=False),
    )
    def sc_kernel(idx_hbm, val_hbm, part_hbm,
                  sidx0, sidx1, sval0, sval1,
                  ridx0, ridx1, rval0, rval1,
                  acc, spacc,
                  sem_in0, sem_in1, sem_sidx0, sem_sidx1, sem_st0, sem_st1):
        cid = lax.axis_index("c")
        sid = lax.axis_index("s")
        wid = sid * NC + cid
        base = pl.multiple_of(wid * chunk, 8)

        def zero_body(i, _):
            acc[pl.ds(i * LANES, LANES)] = jnp.zeros((LANES,), jnp.float32)
            return 0

        lax.fori_loop(0, num_segments // LANES, zero_body, 0)

        sidx_bufs = (sidx0, sidx1)
        sval_bufs = (sval0, sval1)
        ridx_bufs = (ridx0, ridx1)
        rval_bufs = (rval0, rval1)
        in_sems = (sem_in0, sem_in1)
        sidx_sems = (sem_sidx0, sem_sidx1)
        st_sems = (sem_st0, sem_st1)

        def start_in(widx):
            p = widx % 2
            wb = pl.multiple_of(base + widx * w, 8)
            wbr = pl.multiple_of(wb + ws, 8)
            d = [
                pltpu.async_copy(idx_hbm.at[pl.ds(wb, ws)], sidx_bufs[p], sidx_sems[p]),
                pltpu.async_copy(val_hbm.at[pl.ds(wb, ws)], sval_bufs[p], in_sems[p]),
                pltpu.async_copy(idx_hbm.at[pl.ds(wbr, wr)], ridx_bufs[p], in_sems[p]),
                pltpu.async_copy(val_hbm.at[pl.ds(wbr, wr)], rval_bufs[p], in_sems[p]),
            ]
            return d

        @pl.when(sid == 0)
        def _():
            pltpu.sync_copy(acc, spacc)

        pending = start_in(0)
        plsc.subcore_barrier()

        pos0 = lax.iota(jnp.int32, LANES) * sub
        stream_descs = [None] * nwin

        for widx in range(nwin):
            p = widx % 2
            for d in pending:
                d.wait()
            stream_descs[widx] = pltpu.async_copy(
                sval_bufs[p], spacc.at[sidx_bufs[p]], st_sems[p], add=True
            )
            if widx + 1 < nwin:
                if widx - 1 >= 0:
                    stream_descs[widx - 1].wait()
                pending = start_in(widx + 1)
            idxb = ridx_bufs[p]
            valb = rval_bufs[p]

            @plsc.parallel_loop(0, sub, unroll=8)
            def _(t):
                pos = pos0 + t
                iv = plsc.load_gather(idxb, [pos])
                vv = plsc.load_gather(valb, [pos])
                plsc.addupdate_scatter(acc, [iv], vv)

        if nwin >= 2:
            stream_descs[nwin - 2].wait()
        stream_descs[nwin - 1].wait()
        pltpu.sync_copy(acc, part_hbm.at[wid])
        plsc.subcore_barrier()

        @pl.when(sid == 0)
        def _():
            pltpu.sync_copy(spacc, part_hbm.at[NW + cid])

    return sc_kernel(idx, vals)


def _tc_combine(tensor2d, partials):
    def body(t_ref, p_ref, o_ref):
        o_ref[...] = t_ref[...] - jnp.sum(p_ref[...], axis=0, keepdims=True)

    return pl.pallas_call(
        body,
        out_shape=jax.ShapeDtypeStruct(tensor2d.shape, tensor2d.dtype),
    )(tensor2d, partials)


def kernel(tensor, batch_idx, atomic_numbers, element_references):
    num_segments = tensor.shape[0]
    idx = batch_idx.astype(jnp.int32)
    vals = atomic_numbers.astype(jnp.float32)
    partials = _sc_partials(idx, vals, num_segments)
    out2d = _tc_combine(tensor.reshape(1, -1), partials)
    return out2d.reshape(tensor.shape)


# speedup vs baseline: 1.1106x; 1.0377x over previous
import functools
import jax
import jax.numpy as jnp
from jax import lax
from jax.experimental import pallas as pl
from jax.experimental.pallas import tpu as pltpu
from jax.experimental.pallas import tpu_sc as plsc

NC = 2
NS = 16
NW = NC * NS
LANES = 16


def _split_window(chunk):
    for w, ws in ((10000, 1920), (20000, 5120), (8192, 2048), (16, 0)):
        if chunk % w == 0 and (w - ws) % LANES == 0 and ((w - ws) // LANES) % 2 == 1:
            return w, ws
    return chunk, 0


def _sc_partials(idx, vals, num_segments):
    n = vals.shape[0]
    chunk = n // NW
    assert chunk * NW == n
    w, ws = _split_window(chunk)
    nwin = chunk // w
    wr = w - ws
    sub = wr // LANES
    assert num_segments % LANES == 0

    mesh = plsc.VectorSubcoreMesh(
        core_axis_name="c", subcore_axis_name="s", num_cores=NC, num_subcores=NS
    )

    @functools.partial(
        pl.kernel,
        out_type=jax.ShapeDtypeStruct((NW + NC, num_segments), jnp.float32),
        mesh=mesh,
        scratch_types=[
            pltpu.VMEM((ws,), jnp.int32),
            pltpu.VMEM((ws,), jnp.int32),
            pltpu.VMEM((ws,), jnp.float32),
            pltpu.VMEM((ws,), jnp.float32),
            pltpu.VMEM((wr,), jnp.int32),
            pltpu.VMEM((wr,), jnp.int32),
            pltpu.VMEM((wr,), jnp.float32),
            pltpu.VMEM((wr,), jnp.float32),
            pltpu.VMEM((num_segments,), jnp.float32),
            pltpu.VMEM_SHARED((num_segments,), jnp.float32),
            pltpu.SemaphoreType.DMA,
            pltpu.SemaphoreType.DMA,
            pltpu.SemaphoreType.DMA,
            pltpu.SemaphoreType.DMA,
            pltpu.SemaphoreType.DMA,
            pltpu.SemaphoreType.DMA,
        ],
        compiler_params=pltpu.CompilerParams(needs_layout_passes=False),
    )
    def sc_kernel(idx_hbm, val_hbm, part_hbm,
                  sidx0, sidx1, sval0, sval1,
                  ridx0, ridx1, rval0, rval1,
                  acc, spacc,
                  sem_in0, sem_in1, sem_sidx0, sem_sidx1, sem_st0, sem_st1):
        cid = lax.axis_index("c")
        sid = lax.axis_index("s")
        wid = sid * NC + cid
        base = pl.multiple_of(wid * chunk, 8)

        def zero_body(i, _):
            acc[pl.ds(i * LANES, LANES)] = jnp.zeros((LANES,), jnp.float32)
            return 0

        lax.fori_loop(0, num_segments // LANES, zero_body, 0)

        sidx_bufs = (sidx0, sidx1)
        sval_bufs = (sval0, sval1)
        ridx_bufs = (ridx0, ridx1)
        rval_bufs = (rval0, rval1)
        in_sems = (sem_in0, sem_in1)
        sidx_sems = (sem_sidx0, sem_sidx1)
        st_sems = (sem_st0, sem_st1)

        def start_in(widx):
            p = widx % 2
            wb = pl.multiple_of(base + widx * w, 8)
            wbr = pl.multiple_of(wb + ws, 8)
            d = [
                pltpu.async_copy(idx_hbm.at[pl.ds(wb, ws)], sidx_bufs[p], sidx_sems[p]),
                pltpu.async_copy(val_hbm.at[pl.ds(wb, ws)], sval_bufs[p], in_sems[p]),
                pltpu.async_copy(idx_hbm.at[pl.ds(wbr, wr)], ridx_bufs[p], in_sems[p]),
                pltpu.async_copy(val_hbm.at[pl.ds(wbr, wr)], rval_bufs[p], in_sems[p]),
            ]
            return d

        @pl.when(sid == 0)
        def _():
            pltpu.sync_copy(acc, spacc)

        pending = start_in(0)
        plsc.subcore_barrier()

        pos0 = lax.iota(jnp.int32, LANES) * sub
        stream_descs = [None] * nwin

        for widx in range(nwin):
            p = widx % 2
            for d in pending:
                d.wait()
            stream_descs[widx] = pltpu.async_copy(
                sval_bufs[p], spacc.at[sidx_bufs[p]], st_sems[p], add=True
            )
            if widx + 1 < nwin:
                if widx - 1 >= 0:
                    stream_descs[widx - 1].wait()
                pending = start_in(widx + 1)
            idxb = ridx_bufs[p]
            valb = rval_bufs[p]

            @plsc.parallel_loop(0, sub, unroll=8)
            def _(t):
                pos = pos0 + t
                iv = plsc.load_gather(idxb, [pos])
                vv = plsc.load_gather(valb, [pos])
                plsc.addupdate_scatter(acc, [iv], vv)

        if nwin >= 2:
            stream_descs[nwin - 2].wait()
        stream_descs[nwin - 1].wait()
        pltpu.sync_copy(acc, part_hbm.at[wid])
        plsc.subcore_barrier()

        @pl.when(sid == 0)
        def _():
            pltpu.sync_copy(spacc, part_hbm.at[NW + cid])

    return sc_kernel(idx, vals)


def _tc_combine(tensor2d, partials):
    def body(t_ref, p_ref, o_ref):
        o_ref[...] = t_ref[...] - jnp.sum(p_ref[...], axis=0, keepdims=True)

    return pl.pallas_call(
        body,
        out_shape=jax.ShapeDtypeStruct(tensor2d.shape, tensor2d.dtype),
    )(tensor2d, partials)


def kernel(tensor, batch_idx, atomic_numbers, element_references):
    num_segments = tensor.shape[0]
    idx = batch_idx.astype(jnp.int32)
    vals = atomic_numbers.astype(jnp.float32)
    partials = _sc_partials(idx, vals, num_segments)
    out2d = _tc_combine(tensor.reshape(1, -1), partials)
    return out2d.reshape(tensor.shape)


# speedup vs baseline: 1.1568x; 1.0416x over previous
import functools
import jax
import jax.numpy as jnp
from jax import lax
from jax.experimental import pallas as pl
from jax.experimental.pallas import tpu as pltpu
from jax.experimental.pallas import tpu_sc as plsc

NC = 2
NS = 16
NW = NC * NS
LANES = 16


def _split_window(chunk):
    for w, ws in ((10000, 1600), (20000, 5120), (8192, 2048), (16, 0)):
        if chunk % w == 0 and (w - ws) % LANES == 0 and ((w - ws) // LANES) % 2 == 1:
            return w, ws
    return chunk, 0


def _sc_partials(idx, vals, num_segments):
    n = vals.shape[0]
    chunk = n // NW
    assert chunk * NW == n
    w, ws = _split_window(chunk)
    nwin = chunk // w
    wr = w - ws
    sub = wr // LANES
    assert num_segments % LANES == 0

    mesh = plsc.VectorSubcoreMesh(
        core_axis_name="c", subcore_axis_name="s", num_cores=NC, num_subcores=NS
    )

    @functools.partial(
        pl.kernel,
        out_type=jax.ShapeDtypeStruct((NW + NC, num_segments), jnp.float32),
        mesh=mesh,
        scratch_types=[
            pltpu.VMEM((ws,), jnp.int32),
            pltpu.VMEM((ws,), jnp.int32),
            pltpu.VMEM((ws,), jnp.float32),
            pltpu.VMEM((ws,), jnp.float32),
            pltpu.VMEM((wr,), jnp.int32),
            pltpu.VMEM((wr,), jnp.int32),
            pltpu.VMEM((wr,), jnp.float32),
            pltpu.VMEM((wr,), jnp.float32),
            pltpu.VMEM((num_segments,), jnp.float32),
            pltpu.VMEM_SHARED((num_segments,), jnp.float32),
            pltpu.SemaphoreType.DMA,
            pltpu.SemaphoreType.DMA,
            pltpu.SemaphoreType.DMA,
            pltpu.SemaphoreType.DMA,
            pltpu.SemaphoreType.DMA,
            pltpu.SemaphoreType.DMA,
        ],
        compiler_params=pltpu.CompilerParams(needs_layout_passes=False),
    )
    def sc_kernel(idx_hbm, val_hbm, part_hbm,
                  sidx0, sidx1, sval0, sval1,
                  ridx0, ridx1, rval0, rval1,
                  acc, spacc,
                  sem_in0, sem_in1, sem_sidx0, sem_sidx1, sem_st0, sem_st1):
        cid = lax.axis_index("c")
        sid = lax.axis_index("s")
        wid = sid * NC + cid
        base = pl.multiple_of(wid * chunk, 8)

        def zero_body(i, _):
            acc[pl.ds(i * LANES, LANES)] = jnp.zeros((LANES,), jnp.float32)
            return 0

        lax.fori_loop(0, num_segments // LANES, zero_body, 0)

        sidx_bufs = (sidx0, sidx1)
        sval_bufs = (sval0, sval1)
        ridx_bufs = (ridx0, ridx1)
        rval_bufs = (rval0, rval1)
        in_sems = (sem_in0, sem_in1)
        sidx_sems = (sem_sidx0, sem_sidx1)
        st_sems = (sem_st0, sem_st1)

        def start_in(widx):
            p = widx % 2
            wb = pl.multiple_of(base + widx * w, 8)
            wbr = pl.multiple_of(wb + ws, 8)
            d = [
                pltpu.async_copy(idx_hbm.at[pl.ds(wb, ws)], sidx_bufs[p], sidx_sems[p]),
                pltpu.async_copy(val_hbm.at[pl.ds(wb, ws)], sval_bufs[p], in_sems[p]),
                pltpu.async_copy(idx_hbm.at[pl.ds(wbr, wr)], ridx_bufs[p], in_sems[p]),
                pltpu.async_copy(val_hbm.at[pl.ds(wbr, wr)], rval_bufs[p], in_sems[p]),
            ]
            return d

        @pl.when(sid == 0)
        def _():
            pltpu.sync_copy(acc, spacc)

        pending = start_in(0)
        plsc.subcore_barrier()

        pos0 = lax.iota(jnp.int32, LANES) * sub
        stream_descs = [None] * nwin

        for widx in range(nwin):
            p = widx % 2
            for d in pending:
                d.wait()
            stream_descs[widx] = pltpu.async_copy(
                sval_bufs[p], spacc.at[sidx_bufs[p]], st_sems[p], add=True
            )
            if widx + 1 < nwin:
                if widx - 1 >= 0:
                    stream_descs[widx - 1].wait()
                pending = start_in(widx + 1)
            idxb = ridx_bufs[p]
            valb = rval_bufs[p]

            @plsc.parallel_loop(0, sub, unroll=8)
            def _(t):
                pos = pos0 + t
                iv = plsc.load_gather(idxb, [pos])
                vv = plsc.load_gather(valb, [pos])
                plsc.addupdate_scatter(acc, [iv], vv)

        if nwin >= 2:
            stream_descs[nwin - 2].wait()
        stream_descs[nwin - 1].wait()
        pltpu.sync_copy(acc, part_hbm.at[wid])
        plsc.subcore_barrier()

        @pl.when(sid == 0)
        def _():
            pltpu.sync_copy(spacc, part_hbm.at[NW + cid])

    return sc_kernel(idx, vals)


def _tc_combine(tensor2d, partials):
    def body(t_ref, p_ref, o_ref):
        o_ref[...] = t_ref[...] - jnp.sum(p_ref[...], axis=0, keepdims=True)

    return pl.pallas_call(
        body,
        out_shape=jax.ShapeDtypeStruct(tensor2d.shape, tensor2d.dtype),
    )(tensor2d, partials)


def kernel(tensor, batch_idx, atomic_numbers, element_references):
    num_segments = tensor.shape[0]
    idx = batch_idx.astype(jnp.int32)
    vals = atomic_numbers.astype(jnp.float32)
    partials = _sc_partials(idx, vals, num_segments)
    out2d = _tc_combine(tensor.reshape(1, -1), partials)
    return out2d.reshape(tensor.shape)


# speedup vs baseline: 1.2234x; 1.0576x over previous
import functools
import jax
import jax.numpy as jnp
from jax import lax
from jax.experimental import pallas as pl
from jax.experimental.pallas import tpu as pltpu
from jax.experimental.pallas import tpu_sc as plsc

NC = 2
NS = 16
NW = NC * NS
LANES = 16


def _split_window(chunk):
    for w, ws in ((10000, 960), (20000, 5120), (8192, 2048), (16, 0)):
        if chunk % w == 0 and (w - ws) % LANES == 0 and ((w - ws) // LANES) % 2 == 1:
            return w, ws
    return chunk, 0


def _sc_partials(idx, vals, num_segments):
    n = vals.shape[0]
    chunk = n // NW
    assert chunk * NW == n
    w, ws = _split_window(chunk)
    nwin = chunk // w
    wr = w - ws
    sub = wr // LANES
    assert num_segments % LANES == 0

    mesh = plsc.VectorSubcoreMesh(
        core_axis_name="c", subcore_axis_name="s", num_cores=NC, num_subcores=NS
    )

    @functools.partial(
        pl.kernel,
        out_type=jax.ShapeDtypeStruct((NW + NC, num_segments), jnp.float32),
        mesh=mesh,
        scratch_types=[
            pltpu.VMEM((ws,), jnp.int32),
            pltpu.VMEM((ws,), jnp.int32),
            pltpu.VMEM((ws,), jnp.float32),
            pltpu.VMEM((ws,), jnp.float32),
            pltpu.VMEM((wr,), jnp.int32),
            pltpu.VMEM((wr,), jnp.int32),
            pltpu.VMEM((wr,), jnp.float32),
            pltpu.VMEM((wr,), jnp.float32),
            pltpu.VMEM((num_segments,), jnp.float32),
            pltpu.VMEM_SHARED((num_segments,), jnp.float32),
            pltpu.SemaphoreType.DMA,
            pltpu.SemaphoreType.DMA,
            pltpu.SemaphoreType.DMA,
            pltpu.SemaphoreType.DMA,
            pltpu.SemaphoreType.DMA,
            pltpu.SemaphoreType.DMA,
        ],
        compiler_params=pltpu.CompilerParams(needs_layout_passes=False),
    )
    def sc_kernel(idx_hbm, val_hbm, part_hbm,
                  sidx0, sidx1, sval0, sval1,
                  ridx0, ridx1, rval0, rval1,
                  acc, spacc,
                  sem_in0, sem_in1, sem_sidx0, sem_sidx1, sem_st0, sem_st1):
        cid = lax.axis_index("c")
        sid = lax.axis_index("s")
        wid = sid * NC + cid
        base = pl.multiple_of(wid * chunk, 8)

        def zero_body(i, _):
            acc[pl.ds(i * LANES, LANES)] = jnp.zeros((LANES,), jnp.float32)
            return 0

        lax.fori_loop(0, num_segments // LANES, zero_body, 0)

        sidx_bufs = (sidx0, sidx1)
        sval_bufs = (sval0, sval1)
        ridx_bufs = (ridx0, ridx1)
        rval_bufs = (rval0, rval1)
        in_sems = (sem_in0, sem_in1)
        sidx_sems = (sem_sidx0, sem_sidx1)
        st_sems = (sem_st0, sem_st1)

        def start_in(widx):
            p = widx % 2
            wb = pl.multiple_of(base + widx * w, 8)
            wbr = pl.multiple_of(wb + ws, 8)
            d = [
                pltpu.async_copy(idx_hbm.at[pl.ds(wb, ws)], sidx_bufs[p], sidx_sems[p]),
                pltpu.async_copy(val_hbm.at[pl.ds(wb, ws)], sval_bufs[p], in_sems[p]),
                pltpu.async_copy(idx_hbm.at[pl.ds(wbr, wr)], ridx_bufs[p], in_sems[p]),
                pltpu.async_copy(val_hbm.at[pl.ds(wbr, wr)], rval_bufs[p], in_sems[p]),
            ]
            return d

        @pl.when(sid == 0)
        def _():
            pltpu.sync_copy(acc, spacc)

        pending = start_in(0)
        plsc.subcore_barrier()

        pos0 = lax.iota(jnp.int32, LANES) * sub
        stream_descs = [None] * nwin

        for widx in range(nwin):
            p = widx % 2
            for d in pending:
                d.wait()
            stream_descs[widx] = pltpu.async_copy(
                sval_bufs[p], spacc.at[sidx_bufs[p]], st_sems[p], add=True
            )
            if widx + 1 < nwin:
                if widx - 1 >= 0:
                    stream_descs[widx - 1].wait()
                pending = start_in(widx + 1)
            idxb = ridx_bufs[p]
            valb = rval_bufs[p]

            @plsc.parallel_loop(0, sub, unroll=8)
            def _(t):
                pos = pos0 + t
                iv = plsc.load_gather(idxb, [pos])
                vv = plsc.load_gather(valb, [pos])
                plsc.addupdate_scatter(acc, [iv], vv)

        if nwin >= 2:
            stream_descs[nwin - 2].wait()
        stream_descs[nwin - 1].wait()
        pltpu.sync_copy(acc, part_hbm.at[wid])
        plsc.subcore_barrier()

        @pl.when(sid == 0)
        def _():
            pltpu.sync_copy(spacc, part_hbm.at[NW + cid])

    return sc_kernel(idx, vals)


def _tc_combine(tensor2d, partials):
    def body(t_ref, p_ref, o_ref):
        o_ref[...] = t_ref[...] - jnp.sum(p_ref[...], axis=0, keepdims=True)

    return pl.pallas_call(
        body,
        out_shape=jax.ShapeDtypeStruct(tensor2d.shape, tensor2d.dtype),
    )(tensor2d, partials)


def kernel(tensor, batch_idx, atomic_numbers, element_references):
    num_segments = tensor.shape[0]
    idx = batch_idx.astype(jnp.int32)
    vals = atomic_numbers.astype(jnp.float32)
    partials = _sc_partials(idx, vals, num_segments)
    out2d = _tc_combine(tensor.reshape(1, -1), partials)
    return out2d.reshape(tensor.shape)


# speedup vs baseline: 1.2691x; 1.0373x over previous
import functools
import jax
import jax.numpy as jnp
from jax import lax
from jax.experimental import pallas as pl
from jax.experimental.pallas import tpu as pltpu
from jax.experimental.pallas import tpu_sc as plsc

NC = 2
NS = 16
NW = NC * NS
LANES = 16


def _split_window(chunk):
    for w, ws in ((10000, 320), (20000, 5120), (8192, 2048), (16, 0)):
        if chunk % w == 0 and (w - ws) % LANES == 0 and ((w - ws) // LANES) % 2 == 1:
            return w, ws
    return chunk, 0


def _sc_partials(idx, vals, num_segments):
    n = vals.shape[0]
    chunk = n // NW
    assert chunk * NW == n
    w, ws = _split_window(chunk)
    nwin = chunk // w
    wr = w - ws
    sub = wr // LANES
    assert num_segments % LANES == 0

    mesh = plsc.VectorSubcoreMesh(
        core_axis_name="c", subcore_axis_name="s", num_cores=NC, num_subcores=NS
    )

    @functools.partial(
        pl.kernel,
        out_type=jax.ShapeDtypeStruct((NW + NC, num_segments), jnp.float32),
        mesh=mesh,
        scratch_types=[
            pltpu.VMEM((ws,), jnp.int32),
            pltpu.VMEM((ws,), jnp.int32),
            pltpu.VMEM((ws,), jnp.float32),
            pltpu.VMEM((ws,), jnp.float32),
            pltpu.VMEM((wr,), jnp.int32),
            pltpu.VMEM((wr,), jnp.int32),
            pltpu.VMEM((wr,), jnp.float32),
            pltpu.VMEM((wr,), jnp.float32),
            pltpu.VMEM((num_segments,), jnp.float32),
            pltpu.VMEM_SHARED((num_segments,), jnp.float32),
            pltpu.SemaphoreType.DMA,
            pltpu.SemaphoreType.DMA,
            pltpu.SemaphoreType.DMA,
            pltpu.SemaphoreType.DMA,
            pltpu.SemaphoreType.DMA,
            pltpu.SemaphoreType.DMA,
        ],
        compiler_params=pltpu.CompilerParams(needs_layout_passes=False),
    )
    def sc_kernel(idx_hbm, val_hbm, part_hbm,
                  sidx0, sidx1, sval0, sval1,
                  ridx0, ridx1, rval0, rval1,
                  acc, spacc,
                  sem_in0, sem_in1, sem_sidx0, sem_sidx1, sem_st0, sem_st1):
        cid = lax.axis_index("c")
        sid = lax.axis_index("s")
        wid = sid * NC + cid
        base = pl.multiple_of(wid * chunk, 8)

        def zero_body(i, _):
            acc[pl.ds(i * LANES, LANES)] = jnp.zeros((LANES,), jnp.float32)
            return 0

        lax.fori_loop(0, num_segments // LANES, zero_body, 0)

        sidx_bufs = (sidx0, sidx1)
        sval_bufs = (sval0, sval1)
        ridx_bufs = (ridx0, ridx1)
        rval_bufs = (rval0, rval1)
        in_sems = (sem_in0, sem_in1)
        sidx_sems = (sem_sidx0, sem_sidx1)
        st_sems = (sem_st0, sem_st1)

        def start_in(widx):
            p = widx % 2
            wb = pl.multiple_of(base + widx * w, 8)
            wbr = pl.multiple_of(wb + ws, 8)
            d = [
                pltpu.async_copy(idx_hbm.at[pl.ds(wb, ws)], sidx_bufs[p], sidx_sems[p]),
                pltpu.async_copy(val_hbm.at[pl.ds(wb, ws)], sval_bufs[p], in_sems[p]),
                pltpu.async_copy(idx_hbm.at[pl.ds(wbr, wr)], ridx_bufs[p], in_sems[p]),
                pltpu.async_copy(val_hbm.at[pl.ds(wbr, wr)], rval_bufs[p], in_sems[p]),
            ]
            return d

        @pl.when(sid == 0)
        def _():
            pltpu.sync_copy(acc, spacc)

        pending = start_in(0)
        plsc.subcore_barrier()

        pos0 = lax.iota(jnp.int32, LANES) * sub
        stream_descs = [None] * nwin

        for widx in range(nwin):
            p = widx % 2
            for d in pending:
                d.wait()
            stream_descs[widx] = pltpu.async_copy(
                sval_bufs[p], spacc.at[sidx_bufs[p]], st_sems[p], add=True
            )
            if widx + 1 < nwin:
                if widx - 1 >= 0:
                    stream_descs[widx - 1].wait()
                pending = start_in(widx + 1)
            idxb = ridx_bufs[p]
            valb = rval_bufs[p]

            @plsc.parallel_loop(0, sub, unroll=8)
            def _(t):
                pos = pos0 + t
                iv = plsc.load_gather(idxb, [pos])
                vv = plsc.load_gather(valb, [pos])
                plsc.addupdate_scatter(acc, [iv], vv)

        if nwin >= 2:
            stream_descs[nwin - 2].wait()
        stream_descs[nwin - 1].wait()
        pltpu.sync_copy(acc, part_hbm.at[wid])
        plsc.subcore_barrier()

        @pl.when(sid == 0)
        def _():
            pltpu.sync_copy(spacc, part_hbm.at[NW + cid])

    return sc_kernel(idx, vals)


def _tc_combine(tensor2d, partials):
    def body(t_ref, p_ref, o_ref):
        o_ref[...] = t_ref[...] - jnp.sum(p_ref[...], axis=0, keepdims=True)

    return pl.pallas_call(
        body,
        out_shape=jax.ShapeDtypeStruct(tensor2d.shape, tensor2d.dtype),
    )(tensor2d, partials)


def kernel(tensor, batch_idx, atomic_numbers, element_references):
    num_segments = tensor.shape[0]
    idx = batch_idx.astype(jnp.int32)
    vals = atomic_numbers.astype(jnp.float32)
    partials = _sc_partials(idx, vals, num_segments)
    out2d = _tc_combine(tensor.reshape(1, -1), partials)
    return out2d.reshape(tensor.shape)
